# baseline (speedup 1.0000x reference)
import jax
import jax.numpy as jnp
from jax.experimental import pallas as pl
from jax.experimental.pallas import tpu as pltpu


def _fused_body(x_ref, w1t_ref, b1_ref, w2_ref, b2_ref, wfct_ref, bfc_ref,
                colx_ref, w4_ref, b4_ref, w24_ref, bc2_ref,
                out_ref, lg_ref, cl_ref, pt_ref):
    p = pl.program_id(1)
    n = colx_ref.shape[3]

    @pl.when(p == 0)
    def _node():
        x = x_ref[0]
        h = jnp.maximum(
            jax.lax.dot_general(x, w1t_ref[...], (((1,), (1,)), ((), ())),
                                preferred_element_type=jnp.float32)
            + b1_ref[...], 0.0)
        h = jnp.maximum(
            jnp.dot(h, w2_ref[...], preferred_element_type=jnp.float32)
            + b2_ref[...], 0.0)
        lg_ref[...] = (
            jax.lax.dot_general(wfct_ref[...], h, (((1,), (1,)), ((), ())),
                                preferred_element_type=jnp.float32)
            + bfc_ref[...])

    @pl.when(p > 0)
    def _col():
        j = p - 1
        x = colx_ref[0]
        for g in range(2):
            xg = x[4 * g:4 * (g + 1)].reshape(4 * 32, n)
            hg = jnp.maximum(
                jnp.dot(w4_ref[...], xg, preferred_element_type=jnp.float32)
                + b4_ref[...], 0.0)
            clg = (jnp.dot(w24_ref[...], hg,
                           preferred_element_type=jnp.float32)
                   + bc2_ref[0, 0])
            cl_ref[j, 4 * g:4 * (g + 1), :] = clg

    @pl.when(p == 2)
    def _finish():
        row = lg_ref[...]
        m = jnp.max(row)
        e = jnp.exp(row - m)
        nodep = e / jnp.sum(e)
        cl = cl_ref[...].reshape(16, n)
        cm = jnp.max(cl, axis=0, keepdims=True)
        ce = jnp.exp(cl - cm)
        cp = ce / jnp.sum(ce, axis=0, keepdims=True)
        prod = cp * nodep
        pt_ref[...] = prod.T
        for r in range(8):
            out_ref[:, 0, 0, r * 16:(r + 1) * 16] = pt_ref[r::8, :]


@jax.jit
def kernel(node_features, col_features, edge_index, W1, b1, W2, b2, Wfc, bfc,
           Wc1, bc1, Wc2, bc2):
    del edge_index
    B, N, FN = node_features.shape
    K, FC = col_features.shape[2], col_features.shape[3]
    H1 = W1.shape[1]

    eye4 = jnp.eye(4, dtype=jnp.float32)
    W4 = jnp.kron(eye4, Wc1.T)
    W24 = jnp.kron(eye4, Wc2.T)
    b4 = jnp.tile(bc1, 4).reshape(-1, 1)

    colT = jnp.transpose(col_features, (0, 2, 3, 1))

    out = pl.pallas_call(
        _fused_body,
        grid=(B, 3),
        in_specs=[
            pl.BlockSpec((1, N, FN), lambda b, p: (b, 0, 0)),
            pl.BlockSpec((H1, FN), lambda b, p: (0, 0)),
            pl.BlockSpec((1, H1), lambda b, p: (0, 0)),
            pl.BlockSpec((H1, H1), lambda b, p: (0, 0)),
            pl.BlockSpec((1, H1), lambda b, p: (0, 0)),
            pl.BlockSpec((1, H1), lambda b, p: (0, 0)),
            pl.BlockSpec((1, 1), lambda b, p: (0, 0)),
            pl.BlockSpec((1, K // 2, FC, N),
                         lambda b, p: (b, jnp.maximum(p - 1, 0), 0, 0)),
            pl.BlockSpec((64, 128), lambda b, p: (0, 0)),
            pl.BlockSpec((64, 1), lambda b, p: (0, 0)),
            pl.BlockSpec((4, 64), lambda b, p: (0, 0)),
            pl.BlockSpec((1, 1), lambda b, p: (0, 0)),
        ],
        out_specs=pl.BlockSpec((N // 8, 1, 1, 8 * K), lambda b, p: (0, b, 0, 0)),
        out_shape=jax.ShapeDtypeStruct((N // 8, B, 1, 8 * K), jnp.float32),
        scratch_shapes=[
            pltpu.VMEM((1, N), jnp.float32),
            pltpu.VMEM((2, K // 2, N), jnp.float32),
            pltpu.VMEM((N, K), jnp.float32),
        ],
    )(node_features, W1.T, b1.reshape(1, -1), W2, b2.reshape(1, -1), Wfc.T,
      bfc.reshape(1, 1), colT, W4, b4, W24, bc2.reshape(1, 1))

    return jnp.transpose(out, (1, 0, 2, 3)).reshape(B, N * K)


# speedup vs baseline: 1.3502x; 1.3502x over previous
import jax
import jax.numpy as jnp
from jax.experimental import pallas as pl
from jax.experimental.pallas import tpu as pltpu


def _fused_body(x_ref, w1t_ref, b1_ref, w2_ref, b2_ref, wfct_ref, bfc_ref,
                colx_ref, w4_ref, b4_ref, w24_ref, bc2_ref,
                out_ref, lg_ref, cl_ref, pt_ref):
    p = pl.program_id(1)
    n = colx_ref.shape[3]

    @pl.when(p == 0)
    def _node():
        x = x_ref[0]
        h = jnp.maximum(
            jax.lax.dot_general(x, w1t_ref[...], (((1,), (1,)), ((), ())),
                                preferred_element_type=jnp.float32)
            + b1_ref[...], 0.0)
        h = jnp.maximum(
            jnp.dot(h, w2_ref[...], preferred_element_type=jnp.float32)
            + b2_ref[...], 0.0)
        lg_ref[...] = (
            jax.lax.dot_general(wfct_ref[...], h, (((1,), (1,)), ((), ())),
                                preferred_element_type=jnp.float32)
            + bfc_ref[...])

    @pl.when(p > 0)
    def _col():
        j = p - 1
        x = colx_ref[0]
        xg = x.reshape(4 * 32, n)
        hg = jnp.maximum(
            jnp.dot(w4_ref[...], xg, preferred_element_type=jnp.float32)
            + b4_ref[...], 0.0)
        clg = (jnp.dot(w24_ref[...], hg, preferred_element_type=jnp.float32)
               + bc2_ref[0, 0])
        cl_ref[j] = clg

    @pl.when(p == 1)
    def _nodep():
        row = lg_ref[...]
        m = jnp.max(row)
        e = jnp.exp(row - m)
        lg_ref[...] = e / jnp.sum(e)

    @pl.when(p == 4)
    def _finish():
        nodep = lg_ref[...]
        cl = cl_ref[...].reshape(16, n)
        cm = jnp.max(cl, axis=0, keepdims=True)
        ce = jnp.exp(cl - cm)
        cp = ce / jnp.sum(ce, axis=0, keepdims=True)
        prod = cp * nodep
        pt_ref[...] = prod.T
        for r in range(8):
            out_ref[0, :, r * 16:(r + 1) * 16] = pt_ref[r::8, :]


@jax.jit
def kernel(node_features, col_features, edge_index, W1, b1, W2, b2, Wfc, bfc,
           Wc1, bc1, Wc2, bc2):
    del edge_index
    B, N, FN = node_features.shape
    K, FC = col_features.shape[2], col_features.shape[3]
    H1 = W1.shape[1]

    eye4 = jnp.eye(4, dtype=jnp.float32)
    W4 = jnp.kron(eye4, Wc1.T)
    W24 = jnp.kron(eye4, Wc2.T)
    b4 = jnp.tile(bc1, 4).reshape(-1, 1)

    colT = jnp.transpose(col_features, (0, 2, 3, 1))

    out = pl.pallas_call(
        _fused_body,
        grid=(B, 5),
        in_specs=[
            pl.BlockSpec((1, N, FN), lambda b, p: (b, 0, 0)),
            pl.BlockSpec((H1, FN), lambda b, p: (0, 0)),
            pl.BlockSpec((1, H1), lambda b, p: (0, 0)),
            pl.BlockSpec((H1, H1), lambda b, p: (0, 0)),
            pl.BlockSpec((1, H1), lambda b, p: (0, 0)),
            pl.BlockSpec((1, H1), lambda b, p: (0, 0)),
            pl.BlockSpec((1, 1), lambda b, p: (0, 0)),
            pl.BlockSpec((1, K // 4, FC, N),
                         lambda b, p: (b, jnp.maximum(p - 1, 0), 0, 0)),
            pl.BlockSpec((64, 128), lambda b, p: (0, 0)),
            pl.BlockSpec((64, 1), lambda b, p: (0, 0)),
            pl.BlockSpec((4, 64), lambda b, p: (0, 0)),
            pl.BlockSpec((1, 1), lambda b, p: (0, 0)),
        ],
        out_specs=pl.BlockSpec((1, N // 8, 8 * K), lambda b, p: (b, 0, 0)),
        out_shape=jax.ShapeDtypeStruct((B, N // 8, 8 * K), jnp.float32),
        scratch_shapes=[
            pltpu.VMEM((1, N), jnp.float32),
            pltpu.VMEM((4, K // 4, N), jnp.float32),
            pltpu.VMEM((N, K), jnp.float32),
        ],
    )(node_features, W1.T, b1.reshape(1, -1), W2, b2.reshape(1, -1), Wfc.T,
      bfc.reshape(1, 1), colT, W4, b4, W24, bc2.reshape(1, 1))

    return out.reshape(B, N * K)


# speedup vs baseline: 1.3578x; 1.0056x over previous
import jax
import jax.numpy as jnp
from jax.experimental import pallas as pl
from jax.experimental.pallas import tpu as pltpu


def _fused_body(x_ref, w1t_ref, b1_ref, w2_ref, b2_ref, wfct_ref, bfc_ref,
                colx_ref, w4_ref, b4_ref, w24_ref, bc2_ref,
                out_ref, lg_ref, cl_ref, pt_ref, pk_ref):
    b = pl.program_id(0)
    nb = pl.num_programs(0)
    p = pl.program_id(1)
    n = colx_ref.shape[3]

    @pl.when(p == 0)
    def _node():
        x = x_ref[0]
        h = jnp.maximum(
            jax.lax.dot_general(x, w1t_ref[...], (((1,), (1,)), ((), ())),
                                preferred_element_type=jnp.float32)
            + b1_ref[...], 0.0)
        h = jnp.maximum(
            jnp.dot(h, w2_ref[...], preferred_element_type=jnp.float32)
            + b2_ref[...], 0.0)
        lg_ref[...] = (
            jax.lax.dot_general(wfct_ref[...], h, (((1,), (1,)), ((), ())),
                                preferred_element_type=jnp.float32)
            + bfc_ref[...])

    @pl.when(p > 0)
    def _col():
        j = p - 1
        x = colx_ref[0]
        for g in range(2):
            xg = x[4 * g:4 * (g + 1)].reshape(4 * 32, n)
            hg = jnp.maximum(
                jnp.dot(w4_ref[...], xg, preferred_element_type=jnp.float32)
                + b4_ref[...], 0.0)
            clg = (jnp.dot(w24_ref[...], hg,
                           preferred_element_type=jnp.float32)
                   + bc2_ref[0, 0])
            cl_ref[j, 4 * g:4 * (g + 1), :] = clg

    @pl.when(p == 2)
    def _finish():
        row = lg_ref[...]
        m = jnp.max(row)
        e = jnp.exp(row - m)
        nodep = e / jnp.sum(e)
        cl = cl_ref[...].reshape(16, n)
        cm = jnp.max(cl, axis=0, keepdims=True)
        ce = jnp.exp(cl - cm)
        cp = ce / jnp.sum(ce, axis=0, keepdims=True)
        prod = cp * nodep
        pt_ref[...] = prod.T
        for bb in range(4):

            @pl.when(b == bb)
            def _pack():
                for r in range(8):
                    lo = 128 * bb + r * 16
                    pk_ref[:, lo:lo + 16] = pt_ref[r::8, :]

        @pl.when(b == nb - 1)
        def _flush():
            out_ref[...] = pk_ref[...]


@jax.jit
def kernel(node_features, col_features, edge_index, W1, b1, W2, b2, Wfc, bfc,
           Wc1, bc1, Wc2, bc2):
    del edge_index
    B, N, FN = node_features.shape
    K, FC = col_features.shape[2], col_features.shape[3]
    H1 = W1.shape[1]

    eye4 = jnp.eye(4, dtype=jnp.float32)
    W4 = jnp.kron(eye4, Wc1.T)
    W24 = jnp.kron(eye4, Wc2.T)
    b4 = jnp.tile(bc1, 4).reshape(-1, 1)

    colT = jnp.transpose(col_features, (0, 2, 3, 1))

    out = pl.pallas_call(
        _fused_body,
        grid=(B, 3),
        in_specs=[
            pl.BlockSpec((1, N, FN), lambda b, p: (b, 0, 0)),
            pl.BlockSpec((H1, FN), lambda b, p: (0, 0)),
            pl.BlockSpec((1, H1), lambda b, p: (0, 0)),
            pl.BlockSpec((H1, H1), lambda b, p: (0, 0)),
            pl.BlockSpec((1, H1), lambda b, p: (0, 0)),
            pl.BlockSpec((1, H1), lambda b, p: (0, 0)),
            pl.BlockSpec((1, 1), lambda b, p: (0, 0)),
            pl.BlockSpec((1, K // 2, FC, N),
                         lambda b, p: (b, jnp.maximum(p - 1, 0), 0, 0)),
            pl.BlockSpec((64, 128), lambda b, p: (0, 0)),
            pl.BlockSpec((64, 1), lambda b, p: (0, 0)),
            pl.BlockSpec((4, 64), lambda b, p: (0, 0)),
            pl.BlockSpec((1, 1), lambda b, p: (0, 0)),
        ],
        out_specs=pl.BlockSpec((N // 8, B * 8 * K), lambda b, p: (0, 0)),
        out_shape=jax.ShapeDtypeStruct((N // 8, B * 8 * K), jnp.float32),
        scratch_shapes=[
            pltpu.VMEM((1, N), jnp.float32),
            pltpu.VMEM((2, K // 2, N), jnp.float32),
            pltpu.VMEM((N, K), jnp.float32),
            pltpu.VMEM((N // 8, B * 8 * K), jnp.float32),
        ],
    )(node_features, W1.T, b1.reshape(1, -1), W2, b2.reshape(1, -1), Wfc.T,
      bfc.reshape(1, 1), colT, W4, b4, W24, bc2.reshape(1, 1))

    return out.reshape(N // 8, B, 8 * K).swapaxes(0, 1).reshape(B, N * K)


# speedup vs baseline: 1.4270x; 1.0510x over previous
import jax
import jax.numpy as jnp
from jax.experimental import pallas as pl
from jax.experimental.pallas import tpu as pltpu


def _fused_body(x_ref, w1t_ref, b1_ref, w2_ref, b2_ref, wfct_ref, bfc_ref,
                colx_ref, w4_ref, b4_ref, w24_ref, bc2_ref,
                out_ref, lg_ref, cl_ref, pt_ref):
    p = pl.program_id(1)
    n = colx_ref.shape[3]

    @pl.when(p == 0)
    def _node():
        x = x_ref[0]
        h = jnp.maximum(
            jax.lax.dot_general(x, w1t_ref[...], (((1,), (1,)), ((), ())),
                                preferred_element_type=jnp.float32)
            + b1_ref[...], 0.0)
        h = jnp.maximum(
            jnp.dot(h, w2_ref[...], preferred_element_type=jnp.float32)
            + b2_ref[...], 0.0)
        lg_ref[...] = (
            jax.lax.dot_general(wfct_ref[...], h, (((1,), (1,)), ((), ())),
                                preferred_element_type=jnp.float32)
            + bfc_ref[...])

    @pl.when(p > 0)
    def _col():
        j = p - 1
        x = colx_ref[0]
        for g in range(2):
            xg = x[4 * g:4 * (g + 1)].reshape(4 * 32, n)
            hg = jnp.maximum(
                jnp.dot(w4_ref[...], xg, preferred_element_type=jnp.float32)
                + b4_ref[...], 0.0)
            clg = (jnp.dot(w24_ref[...], hg,
                           preferred_element_type=jnp.float32)
                   + bc2_ref[0, 0])
            cl_ref[j, 4 * g:4 * (g + 1), :] = clg

    @pl.when(p == 2)
    def _finish():
        row = lg_ref[...]
        m = jnp.max(row)
        e = jnp.exp(row - m)
        nodep = e / jnp.sum(e)
        cl = cl_ref[...].reshape(16, n)
        cm = jnp.max(cl, axis=0, keepdims=True)
        ce = jnp.exp(cl - cm)
        cp = ce / jnp.sum(ce, axis=0, keepdims=True)
        prod = cp * nodep
        pt_ref[...] = prod.T
        for r in range(8):
            out_ref[0, :, r * 16:(r + 1) * 16] = pt_ref[r::8, :]


@jax.jit
def kernel(node_features, col_features, edge_index, W1, b1, W2, b2, Wfc, bfc,
           Wc1, bc1, Wc2, bc2):
    del edge_index
    B, N, FN = node_features.shape
    K, FC = col_features.shape[2], col_features.shape[3]
    H1 = W1.shape[1]

    eye4 = jnp.eye(4, dtype=jnp.float32)
    W4 = jnp.kron(eye4, Wc1.T)
    W24 = jnp.kron(eye4, Wc2.T)
    b4 = jnp.tile(bc1, 4).reshape(-1, 1)

    colT = jnp.transpose(col_features, (0, 2, 3, 1))

    out = pl.pallas_call(
        _fused_body,
        grid=(B, 3),
        in_specs=[
            pl.BlockSpec((1, N, FN), lambda b, p: (b, 0, 0)),
            pl.BlockSpec((H1, FN), lambda b, p: (0, 0)),
            pl.BlockSpec((1, H1), lambda b, p: (0, 0)),
            pl.BlockSpec((H1, H1), lambda b, p: (0, 0)),
            pl.BlockSpec((1, H1), lambda b, p: (0, 0)),
            pl.BlockSpec((1, H1), lambda b, p: (0, 0)),
            pl.BlockSpec((1, 1), lambda b, p: (0, 0)),
            pl.BlockSpec((1, K // 2, FC, N),
                         lambda b, p: (b, jnp.maximum(p - 1, 0), 0, 0)),
            pl.BlockSpec((64, 128), lambda b, p: (0, 0)),
            pl.BlockSpec((64, 1), lambda b, p: (0, 0)),
            pl.BlockSpec((4, 64), lambda b, p: (0, 0)),
            pl.BlockSpec((1, 1), lambda b, p: (0, 0)),
        ],
        out_specs=pl.BlockSpec((1, N // 8, 8 * K), lambda b, p: (b, 0, 0)),
        out_shape=jax.ShapeDtypeStruct((B, N // 8, 8 * K), jnp.float32),
        scratch_shapes=[
            pltpu.VMEM((1, N), jnp.float32),
            pltpu.VMEM((2, K // 2, N), jnp.float32),
            pltpu.VMEM((N, K), jnp.float32),
        ],
    )(node_features, W1.T, b1.reshape(1, -1), W2, b2.reshape(1, -1), Wfc.T,
      bfc.reshape(1, 1), colT, W4, b4, W24, bc2.reshape(1, 1))

    return out.reshape(B, N * K)
